# baseline (speedup 1.0000x reference)
import functools

import jax
import jax.numpy as jnp
from jax import lax
from jax.experimental import pallas as pl
from jax.experimental.pallas import tpu as pltpu
from jax.experimental.pallas import tpu_sc as plsc

_VOCAB = 100000
_EMBED_DIM = 64
_N_TOKENS = 10
_BATCH = 32
_SEQ = 2048

_CHUNK = 128
_N_CHUNKS = _SEQ // _CHUNK
_EG = _EMBED_DIM // 8

_NTILES = _VOCAB // 8
_BT = 16
_NBLK = _NTILES // _BT
_REM = _NTILES - _NBLK * _BT
_BPT = 25


def _build_pack_kernel():
    mesh = plsc.VectorSubcoreMesh(core_axis_name="c", subcore_axis_name="s")

    @functools.partial(
        pl.kernel,
        mesh=mesh,
        compiler_params=pltpu.CompilerParams(
            use_tc_tiling_on_sc=True, needs_layout_passes=False
        ),
        out_type=jax.ShapeDtypeStruct((_NBLK * 8 + _REM // 2, 8, 128), jnp.float32),
        scratch_types=[
            pltpu.VMEM((_BT, 8, _EMBED_DIM), jnp.float32),
            pltpu.VMEM((_BT, 8, _EMBED_DIM), jnp.float32),
            pltpu.VMEM((8, 8, 128), jnp.float32),
            pltpu.VMEM((8, 8, 128), jnp.float32),
            pltpu.SemaphoreType.DMA,
            pltpu.SemaphoreType.DMA,
            pltpu.SemaphoreType.DMA,
            pltpu.SemaphoreType.DMA,
        ],
    )
    def k(z_hbm, out_hbm, vin0, vin1, vout0, vout1, gsem0, gsem1, wsem0, wsem1):
        wid = lax.axis_index("s") * 2 + lax.axis_index("c")

        def issue_in(b, vin, sem):
            @pl.when(b < _NBLK)
            def _():
                pltpu.async_copy(z_hbm.at[pl.ds(b * _BT, _BT)], vin, sem)

        def wait_in(b, vin, sem):
            pltpu.make_async_copy(z_hbm.at[pl.ds(b * _BT, _BT)], vin, sem).wait()

        def wait_out(b, vout, sem):
            pltpu.make_async_copy(vout, out_hbm.at[pl.ds(b * 8, 8)], sem).wait()

        def repack(vin, vout):
            @pl.loop(0, 8)
            def _(q):
                for r in range(8):
                    for h in range(2):
                        i_loc = 2 * q + (2 * r + h) // 8
                        s_loc = (2 * r + h) % 8
                        for c in range(4):
                            vout[q, r, pl.ds(h * 64 + c * 16, 16)] = vin[
                                i_loc, s_loc, pl.ds(c * 16, 16)
                            ]

        issue_in(wid, vin0, gsem0)

        @pl.loop(0, (_BPT + 1) // 2)
        def _(m):
            b0 = wid + 64 * m
            b1 = b0 + 32
            b2 = b0 + 64

            issue_in(b1, vin1, gsem1)

            @pl.when((b0 < _NBLK) & (m > 0))
            def _():
                wait_out(b0 - 64, vout0, wsem0)

            @pl.when(b0 < _NBLK)
            def _():
                wait_in(b0, vin0, gsem0)
                repack(vin0, vout0)
                pltpu.async_copy(vout0, out_hbm.at[pl.ds(b0 * 8, 8)], wsem0)

            issue_in(b2, vin0, gsem0)

            @pl.when((b1 < _NBLK) & (m > 0))
            def _():
                wait_out(b1 - 64, vout1, wsem1)

            @pl.when(b1 < _NBLK)
            def _():
                wait_in(b1, vin1, gsem1)
                repack(vin1, vout1)
                pltpu.async_copy(vout1, out_hbm.at[pl.ds(b1 * 8, 8)], wsem1)

        @pl.when(wid + 32 * (_BPT - 1) < _NBLK)
        def _():
            wait_out(wid + 32 * (_BPT - 1), vout0, wsem0)

        @pl.when(wid + 32 * (_BPT - 1) >= _NBLK)
        def _():
            wait_out(wid + 32 * (_BPT - 3), vout0, wsem0)

        wait_out(wid + 32 * (_BPT - 2), vout1, wsem1)

        @pl.when(wid == 0)
        def _():
            pltpu.sync_copy(z_hbm.at[pl.ds(_NBLK * _BT, _REM)],
                            vin0.at[pl.ds(0, _REM)])
            for q in range(_REM // 2):
                for r in range(8):
                    for h in range(2):
                        i_loc = 2 * q + (2 * r + h) // 8
                        s_loc = (2 * r + h) % 8
                        for c in range(4):
                            vout0[q, r, pl.ds(h * 64 + c * 16, 16)] = vin0[
                                i_loc, s_loc, pl.ds(c * 16, 16)
                            ]
            pltpu.sync_copy(vout0.at[pl.ds(0, _REM // 2)],
                            out_hbm.at[pl.ds(_NBLK * 8, _REM // 2)])

    return k


def _build_sc_kernel():
    mesh = plsc.VectorSubcoreMesh(core_axis_name="c", subcore_axis_name="s")

    @functools.partial(
        pl.kernel,
        mesh=mesh,
        compiler_params=pltpu.CompilerParams(
            use_tc_tiling_on_sc=False, needs_layout_passes=False
        ),
        out_type=jax.ShapeDtypeStruct(
            (_BATCH, _EG, _N_CHUNKS, 8, _CHUNK), jnp.float32
        ),
        scratch_types=[
            pltpu.VMEM((_N_CHUNKS, _CHUNK), jnp.int32),
            pltpu.VMEM((_CHUNK, _EMBED_DIM), jnp.float32),
            pltpu.VMEM((_CHUNK, _EMBED_DIM), jnp.float32),
            pltpu.VMEM((_EMBED_DIM, _CHUNK), jnp.float32),
            pltpu.VMEM((_EMBED_DIM, _CHUNK), jnp.float32),
            pltpu.VMEM((_N_TOKENS, _EMBED_DIM), jnp.float32),
            pltpu.SemaphoreType.DMA,
            pltpu.SemaphoreType.DMA,
            pltpu.SemaphoreType.DMA,
            pltpu.SemaphoreType.DMA,
        ],
    )
    def k(tok_hbm, table_hbm, learned_hbm, out_hbm,
          idx_v, rows0, rows1, tp0, tp1, learned_v, gsem0, gsem1, wsem0, wsem1):
        wid = lax.axis_index("s") * 2 + lax.axis_index("c")

        pltpu.sync_copy(tok_hbm.at[wid], idx_v)
        pltpu.sync_copy(learned_hbm, learned_v)

        bufs = (rows0, rows1)
        tbufs = (tp0, tp1)
        gsems = (gsem0, gsem1)
        wsems = (wsem0, wsem1)
        gcopies = [None, None]
        wcopies = [[], []]

        iota = lax.iota(jnp.int32, 16)
        rows16 = [iota + t0 * 16 for t0 in range(_CHUNK // 16)]

        def transpose_chunk(buf, tbuf):
            @plsc.parallel_loop(0, _EMBED_DIM, 1, unroll=4)
            def _(e):
                e_splat = jnp.full((16,), e, jnp.int32)
                for t0 in range(_CHUNK // 16):
                    vals = plsc.load_gather(buf, [rows16[t0], e_splat])
                    tbuf[e, pl.ds(t0 * 16, 16)] = vals

        gcopies[0] = pltpu.async_copy(table_hbm.at[idx_v.at[0]], bufs[0], gsems[0])
        for j in range(_N_CHUNKS):
            p = j % 2
            gcopies[p].wait()
            if j + 1 < _N_CHUNKS:
                gcopies[1 - p] = pltpu.async_copy(
                    table_hbm.at[idx_v.at[j + 1]], bufs[1 - p], gsems[1 - p]
                )
            if j == 0:
                for r in range(_N_TOKENS):
                    for c in range(_EMBED_DIM // 16):
                        bufs[p][r, pl.ds(c * 16, 16)] = learned_v[r, pl.ds(c * 16, 16)]
            for cp in wcopies[p]:
                cp.wait()
            wcopies[p] = []
            transpose_chunk(bufs[p], tbufs[p])
            for g in range(_EG):
                wcopies[p].append(
                    pltpu.async_copy(
                        tbufs[p].at[pl.ds(g * 8, 8)],
                        out_hbm.at[wid, g, j],
                        wsems[p],
                    )
                )
        for p in (0, 1):
            for cp in wcopies[p]:
                cp.wait()

    return k


_pack_kernel = _build_pack_kernel()
_sc_kernel = _build_sc_kernel()


@jax.jit
def kernel(tokens, wte_weight, learned_embedding):
    tok = tokens.astype(jnp.int32).reshape(_BATCH, _N_CHUNKS, _CHUNK)
    z = wte_weight.reshape(_NTILES, 8, _EMBED_DIM)
    packed = _pack_kernel(z)
    table = packed.reshape(_VOCAB, _EMBED_DIM)
    out = _sc_kernel(tok, table, learned_embedding)
    return out.transpose(0, 2, 4, 1, 3).reshape(_BATCH, _SEQ, _EMBED_DIM)


# speedup vs baseline: 1.0713x; 1.0713x over previous
import functools

import jax
import jax.numpy as jnp
from jax import lax
from jax.experimental import pallas as pl
from jax.experimental.pallas import tpu as pltpu
from jax.experimental.pallas import tpu_sc as plsc

_VOCAB = 100000
_EMBED_DIM = 64
_N_TOKENS = 10
_BATCH = 32
_SEQ = 2048

_CHUNK = 128
_N_CHUNKS = _SEQ // _CHUNK
_EG = _EMBED_DIM // 8

_NTILES = _VOCAB // 8
_TPW = _NTILES // 32
_REM = _NTILES - 32 * _TPW
_PBT = 30
_NPB = _TPW // _PBT


def _build_pack_kernel():
    mesh = plsc.VectorSubcoreMesh(core_axis_name="c", subcore_axis_name="s")

    @functools.partial(
        pl.kernel,
        mesh=mesh,
        compiler_params=pltpu.CompilerParams(
            use_tc_tiling_on_sc=True, needs_layout_passes=False
        ),
        out_type=jax.ShapeDtypeStruct((_VOCAB // 16, 8, 128), jnp.float32),
        scratch_types=[
            pltpu.VMEM((_PBT, 8, _EMBED_DIM), jnp.float32),
            pltpu.VMEM((_PBT, 8, _EMBED_DIM), jnp.float32),
            pltpu.VMEM((_PBT // 2, 8, 128), jnp.float32),
            pltpu.VMEM((_PBT // 2, 8, 128), jnp.float32),
            pltpu.SemaphoreType.DMA,
            pltpu.SemaphoreType.DMA,
            pltpu.SemaphoreType.DMA,
            pltpu.SemaphoreType.DMA,
        ],
    )
    def k(z_hbm, out_hbm, vin0, vin1, vout0, vout1, gsem0, gsem1, wsem0, wsem1):
        wid = lax.axis_index("s") * 2 + lax.axis_index("c")
        base = wid * _TPW

        vins = (vin0, vin1)
        vouts = (vout0, vout1)
        gsems = (gsem0, gsem1)
        wsems = (wsem0, wsem1)
        gcp = [None, None]
        wcp = [None, None]

        def in_slice(i):
            return z_hbm.at[pl.ds(base + i * _PBT, _PBT)]

        def repack(vin, vout, nq):
            @plsc.parallel_loop(0, nq, 1, unroll=2)
            def _(q):
                for r in range(8):
                    for h in range(2):
                        i_loc = 2 * q + (2 * r + h) // 8
                        s_loc = (2 * r + h) % 8
                        for c in range(4):
                            vout[q, r, pl.ds(h * 64 + c * 16, 16)] = vin[
                                i_loc, s_loc, pl.ds(c * 16, 16)
                            ]

        def out_slice(i):
            return out_hbm.at[pl.ds((base + i * _PBT) // 2, _PBT // 2)]

        def wait_in(i, p):
            pltpu.make_async_copy(in_slice(i), vins[p], gsems[p]).wait()

        def wait_out(i, p):
            pltpu.make_async_copy(vouts[p], out_slice(i), wsems[p]).wait()

        pltpu.async_copy(in_slice(0), vins[0], gsems[0])

        @pl.loop(0, (_NPB - 1) // 2)
        def _(m):
            i0 = 2 * m
            i1 = i0 + 1
            wait_in(i0, 0)
            pltpu.async_copy(in_slice(i1), vins[1], gsems[1])

            @pl.when(m > 0)
            def _():
                wait_out(i0 - 2, 0)

            repack(vins[0], vouts[0], _PBT // 2)
            pltpu.async_copy(vouts[0], out_slice(i0), wsems[0])

            pltpu.async_copy(in_slice(i0 + 2), vins[0], gsems[0])
            wait_in(i1, 1)

            @pl.when(m > 0)
            def _():
                wait_out(i1 - 2, 1)

            repack(vins[1], vouts[1], _PBT // 2)
            pltpu.async_copy(vouts[1], out_slice(i1), wsems[1])

        wait_in(_NPB - 1, 0)
        wait_out(_NPB - 3, 0)
        repack(vins[0], vouts[0], _PBT // 2)
        pltpu.async_copy(vouts[0], out_slice(_NPB - 1), wsems[0])
        wait_out(_NPB - 2, 1)
        wait_out(_NPB - 1, 0)

        @pl.when(wid == 0)
        def _():
            pltpu.sync_copy(z_hbm.at[pl.ds(32 * _TPW, _REM)],
                            vin0.at[pl.ds(0, _REM)])
            repack(vin0, vout0, _REM // 2)
            pltpu.sync_copy(
                vout0.at[pl.ds(0, _REM // 2)],
                out_hbm.at[pl.ds(32 * _TPW // 2, _REM // 2)],
            )

    return k


def _build_sc_kernel():
    mesh = plsc.VectorSubcoreMesh(core_axis_name="c", subcore_axis_name="s")

    @functools.partial(
        pl.kernel,
        mesh=mesh,
        compiler_params=pltpu.CompilerParams(
            use_tc_tiling_on_sc=False, needs_layout_passes=False
        ),
        out_type=jax.ShapeDtypeStruct(
            (_BATCH, _EG, _N_CHUNKS, 8, _CHUNK), jnp.float32
        ),
        scratch_types=[
            pltpu.VMEM((_N_CHUNKS, _CHUNK), jnp.int32),
            pltpu.VMEM((_CHUNK, _EMBED_DIM), jnp.float32),
            pltpu.VMEM((_CHUNK, _EMBED_DIM), jnp.float32),
            pltpu.VMEM((_EMBED_DIM, _CHUNK), jnp.float32),
            pltpu.VMEM((_EMBED_DIM, _CHUNK), jnp.float32),
            pltpu.VMEM((_N_TOKENS, _EMBED_DIM), jnp.float32),
            pltpu.SemaphoreType.DMA,
            pltpu.SemaphoreType.DMA,
            pltpu.SemaphoreType.DMA,
            pltpu.SemaphoreType.DMA,
        ],
    )
    def k(tok_hbm, table_hbm, learned_hbm, out_hbm,
          idx_v, rows0, rows1, tp0, tp1, learned_v, gsem0, gsem1, wsem0, wsem1):
        wid = lax.axis_index("s") * 2 + lax.axis_index("c")

        pltpu.sync_copy(tok_hbm.at[wid], idx_v)
        pltpu.sync_copy(learned_hbm, learned_v)

        bufs = (rows0, rows1)
        tbufs = (tp0, tp1)
        gsems = (gsem0, gsem1)
        wsems = (wsem0, wsem1)
        gcopies = [None, None]
        wcopies = [[], []]

        iota = lax.iota(jnp.int32, 16)
        rows16 = [iota + t0 * 16 for t0 in range(_CHUNK // 16)]

        def transpose_chunk(buf, tbuf):
            @plsc.parallel_loop(0, _EMBED_DIM, 1, unroll=4)
            def _(e):
                e_splat = jnp.full((16,), e, jnp.int32)
                for t0 in range(_CHUNK // 16):
                    vals = plsc.load_gather(buf, [rows16[t0], e_splat])
                    tbuf[e, pl.ds(t0 * 16, 16)] = vals

        gcopies[0] = pltpu.async_copy(table_hbm.at[idx_v.at[0]], bufs[0], gsems[0])
        for j in range(_N_CHUNKS):
            p = j % 2
            gcopies[p].wait()
            if j + 1 < _N_CHUNKS:
                gcopies[1 - p] = pltpu.async_copy(
                    table_hbm.at[idx_v.at[j + 1]], bufs[1 - p], gsems[1 - p]
                )
            if j == 0:
                for r in range(_N_TOKENS):
                    for c in range(_EMBED_DIM // 16):
                        bufs[p][r, pl.ds(c * 16, 16)] = learned_v[r, pl.ds(c * 16, 16)]
            for cp in wcopies[p]:
                cp.wait()
            wcopies[p] = []
            transpose_chunk(bufs[p], tbufs[p])
            for g in range(_EG):
                wcopies[p].append(
                    pltpu.async_copy(
                        tbufs[p].at[pl.ds(g * 8, 8)],
                        out_hbm.at[wid, g, j],
                        wsems[p],
                    )
                )
        for p in (0, 1):
            for cp in wcopies[p]:
                cp.wait()

    return k


_pack_kernel = _build_pack_kernel()
_sc_kernel = _build_sc_kernel()


@jax.jit
def kernel(tokens, wte_weight, learned_embedding):
    tok = tokens.astype(jnp.int32).reshape(_BATCH, _N_CHUNKS, _CHUNK)
    z = wte_weight.reshape(_NTILES, 8, _EMBED_DIM)
    packed = _pack_kernel(z)
    table = packed.reshape(_VOCAB, _EMBED_DIM)
    out = _sc_kernel(tok, table, learned_embedding)
    return out.transpose(0, 2, 4, 1, 3).reshape(_BATCH, _SEQ, _EMBED_DIM)


# speedup vs baseline: 1.1526x; 1.0759x over previous
import functools

import jax
import jax.numpy as jnp
from jax import lax
from jax.experimental import pallas as pl
from jax.experimental.pallas import tpu as pltpu
from jax.experimental.pallas import tpu_sc as plsc

_VOCAB = 100000
_EMBED_DIM = 64
_N_TOKENS = 10
_BATCH = 32
_SEQ = 2048

_CHUNK = 128
_N_CHUNKS = _SEQ // _CHUNK
_GROUP = 4
_N_GROUPS = _N_CHUNKS // _GROUP


def _build_sc_kernel():
    mesh = plsc.VectorSubcoreMesh(core_axis_name="c", subcore_axis_name="s")

    @functools.partial(
        pl.kernel,
        mesh=mesh,
        compiler_params=pltpu.CompilerParams(use_tc_tiling_on_sc=False),
        out_type=jax.ShapeDtypeStruct((_BATCH * _SEQ, _EMBED_DIM), jnp.float32),
        scratch_types=[
            pltpu.VMEM((_N_CHUNKS, _CHUNK), jnp.int32),
            pltpu.VMEM((_GROUP * _CHUNK, _EMBED_DIM), jnp.float32),
            pltpu.VMEM((_GROUP * _CHUNK, _EMBED_DIM), jnp.float32),
            pltpu.VMEM((_N_TOKENS, _EMBED_DIM), jnp.float32),
            pltpu.SemaphoreType.DMA,
            pltpu.SemaphoreType.DMA,
            pltpu.SemaphoreType.DMA,
            pltpu.SemaphoreType.DMA,
        ],
    )
    def k(tok_hbm, table_hbm, learned_hbm, out_hbm,
          idx_v, rows0, rows1, learned_v, gsem0, gsem1, wsem0, wsem1):
        wid = lax.axis_index("s") * 2 + lax.axis_index("c")
        base = wid * _SEQ

        pltpu.sync_copy(tok_hbm.at[wid], idx_v)
        pltpu.sync_copy(learned_hbm, learned_v)

        bufs = (rows0, rows1)
        gsems = (gsem0, gsem1)
        wsems = (wsem0, wsem1)
        gcopies = [[None] * _GROUP, [None] * _GROUP]
        wcopy = [None, None]

        def fire(g, p):
            for t in range(_GROUP):
                gcopies[p][t] = pltpu.async_copy(
                    table_hbm.at[idx_v.at[g * _GROUP + t]],
                    bufs[p].at[pl.ds(t * _CHUNK, _CHUNK)],
                    gsems[p],
                )

        fire(0, 0)
        for g in range(_N_GROUPS):
            p = g % 2
            for t in range(_GROUP):
                gcopies[p][t].wait()
            if g + 1 < _N_GROUPS:
                if wcopy[1 - p] is not None:
                    wcopy[1 - p].wait()
                    wcopy[1 - p] = None
                fire(g + 1, 1 - p)
            if g == 0:
                for r in range(_N_TOKENS):
                    for c in range(_EMBED_DIM // 16):
                        bufs[p][r, pl.ds(c * 16, 16)] = learned_v[r, pl.ds(c * 16, 16)]
            wcopy[p] = pltpu.async_copy(
                bufs[p],
                out_hbm.at[pl.ds(base + g * _GROUP * _CHUNK, _GROUP * _CHUNK)],
                wsems[p],
            )
        for p in (0, 1):
            if wcopy[p] is not None:
                wcopy[p].wait()

    return k


_sc_kernel = _build_sc_kernel()


@jax.jit
def kernel(tokens, wte_weight, learned_embedding):
    tok = tokens.astype(jnp.int32).reshape(_BATCH, _N_CHUNKS, _CHUNK)
    out = _sc_kernel(tok, wte_weight, learned_embedding)
    return out.reshape(_BATCH, _SEQ, _EMBED_DIM)
